# Initial kernel scaffold; baseline (speedup 1.0000x reference)
import jax
import jax.numpy as jnp
from jax.experimental import pallas as pl


def kernel(scores, thresholds, y):
    raise NotImplementedError("write your pallas kernel here")


# speedup vs baseline: 7006.2265x; 7006.2265x over previous
import jax
import jax.numpy as jnp
from jax.experimental import pallas as pl
from jax.experimental.pallas import tpu as pltpu

K = 16
ROWS = 8192
COLS = 8192
BLOCK_ROWS = 256


def _body(thr_ref, y_ref, s_ref, o_ref):
    s = s_ref[...]
    m = jnp.max(s, axis=-1, keepdims=True)
    e = jnp.exp(s - m)
    z = jnp.sum(e, axis=-1, keepdims=True)
    v = e / z
    yq = jnp.full_like(s, y_ref[0, 0])
    for k in range(K - 1):
        yq = jnp.where(v > thr_ref[0, k], y_ref[0, k + 1], yq)
    denom = jnp.maximum(jnp.sum(yq, axis=-1, keepdims=True), 1e-30)
    o_ref[...] = yq / denom


def kernel(scores, thresholds, y):
    orig_shape = scores.shape
    s2 = scores.reshape(ROWS, COLS)
    thr = jnp.pad(thresholds, (0, 1)).reshape(1, K)
    y2 = y.reshape(1, K)
    grid = (ROWS // BLOCK_ROWS,)
    out = pl.pallas_call(
        _body,
        grid=grid,
        in_specs=[
            pl.BlockSpec((1, K), lambda i: (0, 0)),
            pl.BlockSpec((1, K), lambda i: (0, 0)),
            pl.BlockSpec((BLOCK_ROWS, COLS), lambda i: (i, 0)),
        ],
        out_specs=pl.BlockSpec((BLOCK_ROWS, COLS), lambda i: (i, 0)),
        out_shape=jax.ShapeDtypeStruct((ROWS, COLS), jnp.float32),
        compiler_params=pltpu.CompilerParams(
            dimension_semantics=("arbitrary",),
        ),
    )(thr, y2, s2)
    return out.reshape(orig_shape)
